# baseline (speedup 1.0000x reference)
import jax
import jax.numpy as jnp
from jax.experimental import pallas as pl
from jax.experimental.pallas import tpu as pltpu

B, S, T, M = 32, 512, 2048, 80
ROWS = B * T
GR = 8
W = GR * M
G = ROWS // GR
BLK = 512
GRID = G // BLK


def _body(mt_ref, mp_ref, pmp_ref, w8_ref, mwf_ref,
          pt_ref, pp_ref, et_ref, ep_ref,
          ldp_ref, dt_ref, pst_ref, psp_ref, sw_ref,
          out_ref, acc1_ref, acc2_ref, e_ref, sacc_ref):
    i = pl.program_id(0)

    @pl.when(i == 0)
    def _small():
        lane = jax.lax.broadcasted_iota(jnp.int32, (GR, W), 1)
        row = jax.lax.broadcasted_iota(jnp.int32, (GR, W), 0)
        e_ref[...] = ((lane // M) == row).astype(jnp.bfloat16)

        sw = sw_ref[...]
        n_src = jnp.sum(sw)
        s_pitch = jnp.sum((pp_ref[...] - pt_ref[...]) ** 2 * sw)
        s_energy = jnp.sum((ep_ref[...] - et_ref[...]) ** 2 * sw)
        ldt = jnp.log(dt_ref[...].astype(jnp.float32) + 1.0)
        s_dur = jnp.sum((ldp_ref[...] - ldt) ** 2 * sw)
        psp = psp_ref[...]
        pst = pst_ref[...]
        d = psp - pst
        s_mid = jnp.sum(d * d)
        cond = jnp.logical_and((0.0 * psp) > (psp - 0.5), pst != 0.0)
        s_pen = jnp.sum(cond.astype(jnp.float32))
        sacc_ref[0] = s_pitch
        sacc_ref[1] = s_energy
        sacc_ref[2] = s_dur
        sacc_ref[3] = n_src
        sacc_ref[4] = s_mid
        sacc_ref[5] = s_pen
        sacc_ref[6] = jnp.sum(mwf_ref[...])
        acc1_ref[...] = jnp.zeros_like(acc1_ref)
        acc2_ref[...] = jnp.zeros_like(acc2_ref)

    w_flat = jax.lax.dot_general(
        w8_ref[...].astype(jnp.bfloat16), e_ref[...],
        (((1,), (0,)), ((), ())),
        preferred_element_type=jnp.float32)
    mt = mt_ref[...]
    acc1_ref[...] += jnp.abs(mp_ref[...] - mt) * w_flat
    acc2_ref[...] += jnp.abs(pmp_ref[...] - mt) * w_flat

    @pl.when(i == GRID - 1)
    def _final():
        n_mel = sacc_ref[6] * M
        mel_loss = jnp.sum(acc1_ref[...]) / n_mel
        postnet_loss = jnp.sum(acc2_ref[...]) / n_mel
        n_src = sacc_ref[3]
        pitch_loss = sacc_ref[0] / n_src
        energy_loss = sacc_ref[1] / n_src
        dur_loss = sacc_ref[2] / n_src
        pause_loss = (sacc_ref[4] / (B * S) + 100.0 * 0.5 * sacc_ref[5] / B) / S
        pause_w = pause_loss * 0.7
        out_ref[1] = mel_loss
        out_ref[2] = postnet_loss
        out_ref[3] = pitch_loss
        out_ref[4] = energy_loss
        out_ref[5] = dur_loss
        out_ref[6] = pause_w
        out_ref[0] = (mel_loss + postnet_loss + dur_loss + pitch_loss +
                      energy_loss + pause_w)


def kernel(mel_targets, pitch_targets, energy_targets, pause_targets,
           mel_predictions, postnet_mel_predictions, pitch_predictions,
           energy_predictions, log_duration_predictions, pause_predictions,
           duration_targets, src_masks, mel_masks):
    mt2 = mel_targets.reshape(G, W)
    mp2 = mel_predictions.reshape(G, W)
    pmp2 = postnet_mel_predictions.reshape(G, W)
    mwf = jnp.logical_not(mel_masks).astype(jnp.float32)
    w8 = mwf.reshape(G, GR)
    sw = jnp.logical_not(src_masks).astype(jnp.float32).reshape(128, 128)

    def r2(x):
        return x.reshape(128, 128)

    mel_spec = pl.BlockSpec((BLK, W), lambda i: (i, 0))
    w8_spec = pl.BlockSpec((BLK, GR), lambda i: (i, 0))
    small_spec = pl.BlockSpec((128, 128), lambda i: (0, 0))
    mwf_spec = pl.BlockSpec((512, 128), lambda i: (0, 0))

    out = pl.pallas_call(
        _body,
        grid=(GRID,),
        in_specs=[mel_spec, mel_spec, mel_spec, w8_spec, mwf_spec] +
                 [small_spec] * 9,
        out_specs=pl.BlockSpec(memory_space=pltpu.SMEM),
        out_shape=jax.ShapeDtypeStruct((8,), jnp.float32),
        scratch_shapes=[pltpu.VMEM((BLK, W), jnp.float32),
                        pltpu.VMEM((BLK, W), jnp.float32),
                        pltpu.VMEM((GR, W), jnp.bfloat16),
                        pltpu.SMEM((8,), jnp.float32)],
    )(mt2, mp2, pmp2, w8, mwf.reshape(512, 128),
      r2(pitch_targets), r2(pitch_predictions),
      r2(energy_targets), r2(energy_predictions),
      r2(log_duration_predictions), r2(duration_targets),
      r2(pause_targets), r2(pause_predictions), sw)

    return (out[0], out[1], out[2], out[3], out[4], out[5], out[6])


# speedup vs baseline: 1.3865x; 1.3865x over previous
import jax
import jax.numpy as jnp
from jax.experimental import pallas as pl
from jax.experimental.pallas import tpu as pltpu

B, S, T, M = 32, 512, 2048, 80
ROWS = B * T
BLK = 2048
GRID = ROWS // BLK


def _body(mt_ref, mp_ref, pmp_ref, mw_ref, mwf_ref,
          pt_ref, pp_ref, et_ref, ep_ref,
          ldp_ref, dt_ref, pst_ref, psp_ref, sw_ref,
          out_ref, acc1_ref, acc2_ref, sacc_ref):
    i = pl.program_id(0)

    @pl.when(i == 0)
    def _small():
        sw = sw_ref[...]
        n_src = jnp.sum(sw)
        s_pitch = jnp.sum((pp_ref[...] - pt_ref[...]) ** 2 * sw)
        s_energy = jnp.sum((ep_ref[...] - et_ref[...]) ** 2 * sw)
        ldt = jnp.log(dt_ref[...].astype(jnp.float32) + 1.0)
        s_dur = jnp.sum((ldp_ref[...] - ldt) ** 2 * sw)
        psp = psp_ref[...]
        pst = pst_ref[...]
        d = psp - pst
        s_mid = jnp.sum(d * d)
        cond = jnp.logical_and((0.0 * psp) > (psp - 0.5), pst != 0.0)
        s_pen = jnp.sum(cond.astype(jnp.float32))
        sacc_ref[0] = s_pitch
        sacc_ref[1] = s_energy
        sacc_ref[2] = s_dur
        sacc_ref[3] = n_src
        sacc_ref[4] = s_mid
        sacc_ref[5] = s_pen
        sacc_ref[6] = jnp.sum(mwf_ref[...])
        acc1_ref[...] = jnp.zeros_like(acc1_ref)
        acc2_ref[...] = jnp.zeros_like(acc2_ref)

    w2 = mw_ref[0].astype(jnp.bfloat16)
    mt = mt_ref[...]
    a1 = jnp.abs(mp_ref[...] - mt).astype(jnp.bfloat16)
    a2 = jnp.abs(pmp_ref[...] - mt).astype(jnp.bfloat16)
    acc1_ref[...] += jax.lax.dot_general(
        w2, a1, (((1,), (0,)), ((), ())),
        preferred_element_type=jnp.float32)
    acc2_ref[...] += jax.lax.dot_general(
        w2, a2, (((1,), (0,)), ((), ())),
        preferred_element_type=jnp.float32)

    @pl.when(i == GRID - 1)
    def _final():
        n_mel = sacc_ref[6] * M
        mel_loss = jnp.sum(acc1_ref[...]) / n_mel
        postnet_loss = jnp.sum(acc2_ref[...]) / n_mel
        n_src = sacc_ref[3]
        pitch_loss = sacc_ref[0] / n_src
        energy_loss = sacc_ref[1] / n_src
        dur_loss = sacc_ref[2] / n_src
        pause_loss = (sacc_ref[4] / (B * S) + 100.0 * 0.5 * sacc_ref[5] / B) / S
        pause_w = pause_loss * 0.7
        out_ref[1] = mel_loss
        out_ref[2] = postnet_loss
        out_ref[3] = pitch_loss
        out_ref[4] = energy_loss
        out_ref[5] = dur_loss
        out_ref[6] = pause_w
        out_ref[0] = (mel_loss + postnet_loss + dur_loss + pitch_loss +
                      energy_loss + pause_w)


def kernel(mel_targets, pitch_targets, energy_targets, pause_targets,
           mel_predictions, postnet_mel_predictions, pitch_predictions,
           energy_predictions, log_duration_predictions, pause_predictions,
           duration_targets, src_masks, mel_masks):
    mt2 = mel_targets.reshape(ROWS, M)
    mp2 = mel_predictions.reshape(ROWS, M)
    pmp2 = postnet_mel_predictions.reshape(ROWS, M)
    mwf = jnp.logical_not(mel_masks).astype(jnp.float32)
    mw = mwf.reshape(GRID, 1, BLK)
    sw = jnp.logical_not(src_masks).astype(jnp.float32).reshape(128, 128)

    def r2(x):
        return x.reshape(128, 128)

    mel_spec = pl.BlockSpec((BLK, M), lambda i: (i, 0))
    mw_spec = pl.BlockSpec((1, 1, BLK), lambda i: (i, 0, 0))
    small_spec = pl.BlockSpec((128, 128), lambda i: (0, 0))
    mwf_spec = pl.BlockSpec((512, 128), lambda i: (0, 0))

    out = pl.pallas_call(
        _body,
        grid=(GRID,),
        in_specs=[mel_spec, mel_spec, mel_spec, mw_spec, mwf_spec] +
                 [small_spec] * 9,
        out_specs=pl.BlockSpec(memory_space=pltpu.SMEM),
        out_shape=jax.ShapeDtypeStruct((8,), jnp.float32),
        scratch_shapes=[pltpu.VMEM((1, M), jnp.float32),
                        pltpu.VMEM((1, M), jnp.float32),
                        pltpu.SMEM((8,), jnp.float32)],
    )(mt2, mp2, pmp2, mw, mwf.reshape(512, 128),
      r2(pitch_targets), r2(pitch_predictions),
      r2(energy_targets), r2(energy_predictions),
      r2(log_duration_predictions), r2(duration_targets),
      r2(pause_targets), r2(pause_predictions), sw)

    return (out[0], out[1], out[2], out[3], out[4], out[5], out[6])


# speedup vs baseline: 1.4017x; 1.0110x over previous
import jax
import jax.numpy as jnp
from jax.experimental import pallas as pl
from jax.experimental.pallas import tpu as pltpu

B, S, T, M = 32, 512, 2048, 80


def _body(mt_ref, mp_ref, pmp_ref, mw_ref,
          pt_ref, pp_ref, et_ref, ep_ref,
          ldp_ref, dt_ref, pst_ref, psp_ref, sw_ref,
          out_ref, acc1_ref, acc2_ref, sacc_ref):
    i = pl.program_id(0)

    @pl.when(i == 0)
    def _small():
        sw = sw_ref[...]
        n_src = jnp.sum(sw)
        s_pitch = jnp.sum((pp_ref[...] - pt_ref[...]) ** 2 * sw)
        s_energy = jnp.sum((ep_ref[...] - et_ref[...]) ** 2 * sw)
        ldt = jnp.log(dt_ref[...].astype(jnp.float32) + 1.0)
        s_dur = jnp.sum((ldp_ref[...] - ldt) ** 2 * sw)
        psp = psp_ref[...]
        pst = pst_ref[...]
        d = psp - pst
        s_mid = jnp.sum(d * d)
        cond = jnp.logical_and((0.0 * psp) > (psp - 0.5), pst != 0.0)
        s_pen = jnp.sum(cond.astype(jnp.float32))
        sacc_ref[0] = s_pitch
        sacc_ref[1] = s_energy
        sacc_ref[2] = s_dur
        sacc_ref[3] = n_src
        sacc_ref[4] = s_mid
        sacc_ref[5] = s_pen
        sacc_ref[6] = jnp.sum(mw_ref[...].astype(jnp.float32))
        acc1_ref[...] = jnp.zeros_like(acc1_ref)
        acc2_ref[...] = jnp.zeros_like(acc2_ref)

    onehot = (jax.lax.broadcasted_iota(jnp.int32, (1, B), 1) == i
              ).astype(jnp.bfloat16)
    w2 = jax.lax.dot_general(
        onehot, mw_ref[...], (((1,), (0,)), ((), ())),
        preferred_element_type=jnp.float32
        ).astype(jnp.bfloat16)
    mt = mt_ref[0]
    a1 = jnp.abs(mp_ref[0] - mt).astype(jnp.bfloat16)
    a2 = jnp.abs(pmp_ref[0] - mt).astype(jnp.bfloat16)
    acc1_ref[...] += jax.lax.dot_general(
        w2, a1, (((1,), (0,)), ((), ())),
        preferred_element_type=jnp.float32)
    acc2_ref[...] += jax.lax.dot_general(
        w2, a2, (((1,), (0,)), ((), ())),
        preferred_element_type=jnp.float32)

    @pl.when(i == B - 1)
    def _final():
        n_mel = sacc_ref[6] * M
        mel_loss = jnp.sum(acc1_ref[...]) / n_mel
        postnet_loss = jnp.sum(acc2_ref[...]) / n_mel
        n_src = sacc_ref[3]
        pitch_loss = sacc_ref[0] / n_src
        energy_loss = sacc_ref[1] / n_src
        dur_loss = sacc_ref[2] / n_src
        pause_loss = (sacc_ref[4] / (B * S) + 100.0 * 0.5 * sacc_ref[5] / B) / S
        pause_w = pause_loss * 0.7
        out_ref[1] = mel_loss
        out_ref[2] = postnet_loss
        out_ref[3] = pitch_loss
        out_ref[4] = energy_loss
        out_ref[5] = dur_loss
        out_ref[6] = pause_w
        out_ref[0] = (mel_loss + postnet_loss + dur_loss + pitch_loss +
                      energy_loss + pause_w)


def kernel(mel_targets, pitch_targets, energy_targets, pause_targets,
           mel_predictions, postnet_mel_predictions, pitch_predictions,
           energy_predictions, log_duration_predictions, pause_predictions,
           duration_targets, src_masks, mel_masks):
    mw = jnp.logical_not(mel_masks).astype(jnp.bfloat16)
    sw = jnp.logical_not(src_masks).astype(jnp.float32)

    mel_spec = pl.BlockSpec((1, T, M), lambda i: (i, 0, 0))
    full2d = pl.BlockSpec((B, T), lambda i: (0, 0))
    small_spec = pl.BlockSpec((B, S), lambda i: (0, 0))

    out = pl.pallas_call(
        _body,
        grid=(B,),
        in_specs=[mel_spec, mel_spec, mel_spec, full2d] + [small_spec] * 9,
        out_specs=pl.BlockSpec(memory_space=pltpu.SMEM),
        out_shape=jax.ShapeDtypeStruct((8,), jnp.float32),
        scratch_shapes=[pltpu.VMEM((1, M), jnp.float32),
                        pltpu.VMEM((1, M), jnp.float32),
                        pltpu.SMEM((8,), jnp.float32)],
    )(mel_targets, mel_predictions, postnet_mel_predictions, mw,
      pitch_targets, pitch_predictions,
      energy_targets, energy_predictions,
      log_duration_predictions, duration_targets,
      pause_targets, pause_predictions, sw)

    return (out[0], out[1], out[2], out[3], out[4], out[5], out[6])


# speedup vs baseline: 4.8535x; 3.4627x over previous
import jax
import jax.numpy as jnp
from jax.experimental import pallas as pl
from jax.experimental.pallas import tpu as pltpu

B, S, T, M = 32, 512, 2048, 80


def _body(mt_ref, mp_ref, pmp_ref, mw_ref,
          pt_ref, pp_ref, et_ref, ep_ref,
          ldp_ref, dt_ref, pst_ref, psp_ref, sw_ref,
          out_ref, acc1_ref, acc2_ref, sacc_ref):
    i = pl.program_id(0)

    @pl.when(i == 0)
    def _small():
        sw = sw_ref[...]
        n_src = jnp.sum(sw)
        s_pitch = jnp.sum((pp_ref[...] - pt_ref[...]) ** 2 * sw)
        s_energy = jnp.sum((ep_ref[...] - et_ref[...]) ** 2 * sw)
        ldt = jnp.log(dt_ref[...].astype(jnp.float32) + 1.0)
        s_dur = jnp.sum((ldp_ref[...] - ldt) ** 2 * sw)
        psp = psp_ref[...]
        pst = pst_ref[...]
        d = psp - pst
        s_mid = jnp.sum(d * d)
        cond = jnp.logical_and((0.0 * psp) > (psp - 0.5), pst != 0.0)
        s_pen = jnp.sum(cond.astype(jnp.float32))
        sacc_ref[0] = s_pitch
        sacc_ref[1] = s_energy
        sacc_ref[2] = s_dur
        sacc_ref[3] = n_src
        sacc_ref[4] = s_mid
        sacc_ref[5] = s_pen
        sacc_ref[6] = jnp.sum(mw_ref[...].astype(jnp.float32))
        acc1_ref[...] = jnp.zeros_like(acc1_ref)
        acc2_ref[...] = jnp.zeros_like(acc2_ref)

    onehot = (jax.lax.broadcasted_iota(jnp.int32, (1, B), 1) == i
              ).astype(jnp.bfloat16)
    w2 = jax.lax.dot_general(
        onehot, mw_ref[...], (((1,), (0,)), ((), ())),
        preferred_element_type=jnp.float32)
    mt = mt_ref[0]
    cs1 = jnp.sum(jnp.abs(mp_ref[0] - mt), axis=0, keepdims=True)
    cs2 = jnp.sum(jnp.abs(pmp_ref[0] - mt), axis=0, keepdims=True)
    acc1_ref[...] += cs1 * w2
    acc2_ref[...] += cs2 * w2

    @pl.when(i == B - 1)
    def _final():
        n_mel = sacc_ref[6] * M
        mel_loss = jnp.sum(acc1_ref[...]) / n_mel
        postnet_loss = jnp.sum(acc2_ref[...]) / n_mel
        n_src = sacc_ref[3]
        pitch_loss = sacc_ref[0] / n_src
        energy_loss = sacc_ref[1] / n_src
        dur_loss = sacc_ref[2] / n_src
        pause_loss = (sacc_ref[4] / (B * S) + 100.0 * 0.5 * sacc_ref[5] / B) / S
        pause_w = pause_loss * 0.7
        out_ref[1] = mel_loss
        out_ref[2] = postnet_loss
        out_ref[3] = pitch_loss
        out_ref[4] = energy_loss
        out_ref[5] = dur_loss
        out_ref[6] = pause_w
        out_ref[0] = (mel_loss + postnet_loss + dur_loss + pitch_loss +
                      energy_loss + pause_w)


def kernel(mel_targets, pitch_targets, energy_targets, pause_targets,
           mel_predictions, postnet_mel_predictions, pitch_predictions,
           energy_predictions, log_duration_predictions, pause_predictions,
           duration_targets, src_masks, mel_masks):
    mt3 = jnp.transpose(mel_targets, (0, 2, 1))
    mp3 = jnp.transpose(mel_predictions, (0, 2, 1))
    pmp3 = jnp.transpose(postnet_mel_predictions, (0, 2, 1))
    mw = jnp.logical_not(mel_masks).astype(jnp.bfloat16)
    sw = jnp.logical_not(src_masks).astype(jnp.float32)

    mel_spec = pl.BlockSpec((1, M, T), lambda i: (i, 0, 0))
    full2d = pl.BlockSpec((B, T), lambda i: (0, 0))
    small_spec = pl.BlockSpec((B, S), lambda i: (0, 0))

    out = pl.pallas_call(
        _body,
        grid=(B,),
        in_specs=[mel_spec, mel_spec, mel_spec, full2d] + [small_spec] * 9,
        out_specs=pl.BlockSpec(memory_space=pltpu.SMEM),
        out_shape=jax.ShapeDtypeStruct((8,), jnp.float32),
        scratch_shapes=[pltpu.VMEM((1, T), jnp.float32),
                        pltpu.VMEM((1, T), jnp.float32),
                        pltpu.SMEM((8,), jnp.float32)],
    )(mt3, mp3, pmp3, mw,
      pitch_targets, pitch_predictions,
      energy_targets, energy_predictions,
      log_duration_predictions, duration_targets,
      pause_targets, pause_predictions, sw)

    return (out[0], out[1], out[2], out[3], out[4], out[5], out[6])


# speedup vs baseline: 6.1950x; 1.2764x over previous
import jax
import jax.numpy as jnp
from jax.experimental import pallas as pl
from jax.experimental.pallas import tpu as pltpu

B, S, T, M = 32, 512, 2048, 80
BB = 2


def _body(mt_ref, mp_ref, pmp_ref, mw_ref,
          pt_ref, pp_ref, et_ref, ep_ref,
          ldp_ref, dt_ref, pst_ref, psp_ref, sw_ref,
          out_ref, acc1_ref, acc2_ref, sacc_ref):
    i = pl.program_id(0)

    @pl.when(i == 0)
    def _small():
        sw = sw_ref[...]
        n_src = jnp.sum(sw)
        s_pitch = jnp.sum((pp_ref[...] - pt_ref[...]) ** 2 * sw)
        s_energy = jnp.sum((ep_ref[...] - et_ref[...]) ** 2 * sw)
        ldt = jnp.log(dt_ref[...].astype(jnp.float32) + 1.0)
        s_dur = jnp.sum((ldp_ref[...] - ldt) ** 2 * sw)
        psp = psp_ref[...]
        pst = pst_ref[...]
        d = psp - pst
        s_mid = jnp.sum(d * d)
        cond = jnp.logical_and((0.0 * psp) > (psp - 0.5), pst != 0.0)
        s_pen = jnp.sum(cond.astype(jnp.float32))
        sacc_ref[0] = s_pitch
        sacc_ref[1] = s_energy
        sacc_ref[2] = s_dur
        sacc_ref[3] = n_src
        sacc_ref[4] = s_mid
        sacc_ref[5] = s_pen
        sacc_ref[6] = jnp.sum(mw_ref[...].astype(jnp.float32))
        acc1_ref[...] = jnp.zeros_like(acc1_ref)
        acc2_ref[...] = jnp.zeros_like(acc2_ref)

    for j in range(BB):
        onehot = (jax.lax.broadcasted_iota(jnp.int32, (1, B), 1) == i * BB + j
                  ).astype(jnp.bfloat16)
        w2 = jax.lax.dot_general(
            onehot, mw_ref[...], (((1,), (0,)), ((), ())),
            preferred_element_type=jnp.float32)
        mt = mt_ref[j]
        cs1 = jnp.sum(jnp.abs(mp_ref[j] - mt), axis=0, keepdims=True)
        cs2 = jnp.sum(jnp.abs(pmp_ref[j] - mt), axis=0, keepdims=True)
        acc1_ref[...] += cs1 * w2
        acc2_ref[...] += cs2 * w2

    @pl.when(i == B // BB - 1)
    def _final():
        n_mel = sacc_ref[6] * M
        mel_loss = jnp.sum(acc1_ref[...]) / n_mel
        postnet_loss = jnp.sum(acc2_ref[...]) / n_mel
        n_src = sacc_ref[3]
        pitch_loss = sacc_ref[0] / n_src
        energy_loss = sacc_ref[1] / n_src
        dur_loss = sacc_ref[2] / n_src
        pause_loss = (sacc_ref[4] / (B * S) + 100.0 * 0.5 * sacc_ref[5] / B) / S
        pause_w = pause_loss * 0.7
        out_ref[1] = mel_loss
        out_ref[2] = postnet_loss
        out_ref[3] = pitch_loss
        out_ref[4] = energy_loss
        out_ref[5] = dur_loss
        out_ref[6] = pause_w
        out_ref[0] = (mel_loss + postnet_loss + dur_loss + pitch_loss +
                      energy_loss + pause_w)


def kernel(mel_targets, pitch_targets, energy_targets, pause_targets,
           mel_predictions, postnet_mel_predictions, pitch_predictions,
           energy_predictions, log_duration_predictions, pause_predictions,
           duration_targets, src_masks, mel_masks):
    mt3 = jnp.transpose(mel_targets, (0, 2, 1))
    mp3 = jnp.transpose(mel_predictions, (0, 2, 1))
    pmp3 = jnp.transpose(postnet_mel_predictions, (0, 2, 1))
    mw = jnp.logical_not(mel_masks).astype(jnp.bfloat16)
    sw = jnp.logical_not(src_masks).astype(jnp.float32)

    mel_spec = pl.BlockSpec((BB, M, T), lambda i: (i, 0, 0))
    full2d = pl.BlockSpec((B, T), lambda i: (0, 0))
    small_spec = pl.BlockSpec((B, S), lambda i: (0, 0))

    out = pl.pallas_call(
        _body,
        grid=(B // BB,),
        in_specs=[mel_spec, mel_spec, mel_spec, full2d] + [small_spec] * 9,
        out_specs=pl.BlockSpec(memory_space=pltpu.SMEM),
        out_shape=jax.ShapeDtypeStruct((8,), jnp.float32),
        scratch_shapes=[pltpu.VMEM((1, T), jnp.float32),
                        pltpu.VMEM((1, T), jnp.float32),
                        pltpu.SMEM((8,), jnp.float32)],
    )(mt3, mp3, pmp3, mw,
      pitch_targets, pitch_predictions,
      energy_targets, energy_predictions,
      log_duration_predictions, duration_targets,
      pause_targets, pause_predictions, sw)

    return (out[0], out[1], out[2], out[3], out[4], out[5], out[6])


# speedup vs baseline: 6.8211x; 1.1011x over previous
import jax
import jax.numpy as jnp
from jax.experimental import pallas as pl
from jax.experimental.pallas import tpu as pltpu

B, S, T, M = 32, 512, 2048, 80
BB = 4


def _body(mt_ref, mp_ref, pmp_ref, mw_ref,
          pt_ref, pp_ref, et_ref, ep_ref,
          ldp_ref, dt_ref, pst_ref, psp_ref, sw_ref,
          out_ref, acc1_ref, acc2_ref, sacc_ref):
    i = pl.program_id(0)

    @pl.when(i == 0)
    def _small():
        sw = sw_ref[...]
        n_src = jnp.sum(sw)
        s_pitch = jnp.sum((pp_ref[...] - pt_ref[...]) ** 2 * sw)
        s_energy = jnp.sum((ep_ref[...] - et_ref[...]) ** 2 * sw)
        ldt = jnp.log(dt_ref[...].astype(jnp.float32) + 1.0)
        s_dur = jnp.sum((ldp_ref[...] - ldt) ** 2 * sw)
        psp = psp_ref[...]
        pst = pst_ref[...]
        d = psp - pst
        s_mid = jnp.sum(d * d)
        cond = jnp.logical_and((0.0 * psp) > (psp - 0.5), pst != 0.0)
        s_pen = jnp.sum(cond.astype(jnp.float32))
        sacc_ref[0] = s_pitch
        sacc_ref[1] = s_energy
        sacc_ref[2] = s_dur
        sacc_ref[3] = n_src
        sacc_ref[4] = s_mid
        sacc_ref[5] = s_pen
        sacc_ref[6] = jnp.sum(mw_ref[...].astype(jnp.float32))
        acc1_ref[...] = jnp.zeros_like(acc1_ref)
        acc2_ref[...] = jnp.zeros_like(acc2_ref)

    for j in range(BB):
        onehot = (jax.lax.broadcasted_iota(jnp.int32, (1, B), 1) == i * BB + j
                  ).astype(jnp.bfloat16)
        w2 = jax.lax.dot_general(
            onehot, mw_ref[...], (((1,), (0,)), ((), ())),
            preferred_element_type=jnp.float32)
        mt = mt_ref[j]
        cs1 = jnp.sum(jnp.abs(mp_ref[j] - mt), axis=0, keepdims=True)
        cs2 = jnp.sum(jnp.abs(pmp_ref[j] - mt), axis=0, keepdims=True)
        acc1_ref[...] += cs1 * w2
        acc2_ref[...] += cs2 * w2

    @pl.when(i == B // BB - 1)
    def _final():
        n_mel = sacc_ref[6] * M
        mel_loss = jnp.sum(acc1_ref[...]) / n_mel
        postnet_loss = jnp.sum(acc2_ref[...]) / n_mel
        n_src = sacc_ref[3]
        pitch_loss = sacc_ref[0] / n_src
        energy_loss = sacc_ref[1] / n_src
        dur_loss = sacc_ref[2] / n_src
        pause_loss = (sacc_ref[4] / (B * S) + 100.0 * 0.5 * sacc_ref[5] / B) / S
        pause_w = pause_loss * 0.7
        out_ref[1] = mel_loss
        out_ref[2] = postnet_loss
        out_ref[3] = pitch_loss
        out_ref[4] = energy_loss
        out_ref[5] = dur_loss
        out_ref[6] = pause_w
        out_ref[0] = (mel_loss + postnet_loss + dur_loss + pitch_loss +
                      energy_loss + pause_w)


def kernel(mel_targets, pitch_targets, energy_targets, pause_targets,
           mel_predictions, postnet_mel_predictions, pitch_predictions,
           energy_predictions, log_duration_predictions, pause_predictions,
           duration_targets, src_masks, mel_masks):
    mt3 = jnp.transpose(mel_targets, (0, 2, 1))
    mp3 = jnp.transpose(mel_predictions, (0, 2, 1))
    pmp3 = jnp.transpose(postnet_mel_predictions, (0, 2, 1))
    mw = jnp.logical_not(mel_masks).astype(jnp.bfloat16)
    sw = jnp.logical_not(src_masks).astype(jnp.float32)

    mel_spec = pl.BlockSpec((BB, M, T), lambda i: (i, 0, 0))
    full2d = pl.BlockSpec((B, T), lambda i: (0, 0))
    small_spec = pl.BlockSpec((B, S), lambda i: (0, 0))

    out = pl.pallas_call(
        _body,
        grid=(B // BB,),
        in_specs=[mel_spec, mel_spec, mel_spec, full2d] + [small_spec] * 9,
        out_specs=pl.BlockSpec(memory_space=pltpu.SMEM),
        out_shape=jax.ShapeDtypeStruct((8,), jnp.float32),
        scratch_shapes=[pltpu.VMEM((1, T), jnp.float32),
                        pltpu.VMEM((1, T), jnp.float32),
                        pltpu.SMEM((8,), jnp.float32)],
    )(mt3, mp3, pmp3, mw,
      pitch_targets, pitch_predictions,
      energy_targets, energy_predictions,
      log_duration_predictions, duration_targets,
      pause_targets, pause_predictions, sw)

    return (out[0], out[1], out[2], out[3], out[4], out[5], out[6])


# speedup vs baseline: 6.8336x; 1.0018x over previous
import jax
import jax.numpy as jnp
from jax.experimental import pallas as pl
from jax.experimental.pallas import tpu as pltpu

B, S, T, M = 32, 512, 2048, 80
BB = 8


def _body(mt_ref, mp_ref, pmp_ref, mw_ref,
          pt_ref, pp_ref, et_ref, ep_ref,
          ldp_ref, dt_ref, pst_ref, psp_ref, sw_ref,
          out_ref, acc1_ref, acc2_ref, sacc_ref):
    i = pl.program_id(0)

    @pl.when(i == 0)
    def _small():
        sw = sw_ref[...]
        n_src = jnp.sum(sw)
        s_pitch = jnp.sum((pp_ref[...] - pt_ref[...]) ** 2 * sw)
        s_energy = jnp.sum((ep_ref[...] - et_ref[...]) ** 2 * sw)
        ldt = jnp.log(dt_ref[...].astype(jnp.float32) + 1.0)
        s_dur = jnp.sum((ldp_ref[...] - ldt) ** 2 * sw)
        psp = psp_ref[...]
        pst = pst_ref[...]
        d = psp - pst
        s_mid = jnp.sum(d * d)
        cond = jnp.logical_and((0.0 * psp) > (psp - 0.5), pst != 0.0)
        s_pen = jnp.sum(cond.astype(jnp.float32))
        sacc_ref[0] = s_pitch
        sacc_ref[1] = s_energy
        sacc_ref[2] = s_dur
        sacc_ref[3] = n_src
        sacc_ref[4] = s_mid
        sacc_ref[5] = s_pen
        sacc_ref[6] = jnp.sum(mw_ref[...].astype(jnp.float32))
        acc1_ref[...] = jnp.zeros_like(acc1_ref)
        acc2_ref[...] = jnp.zeros_like(acc2_ref)

    for j in range(BB):
        onehot = (jax.lax.broadcasted_iota(jnp.int32, (1, B), 1) == i * BB + j
                  ).astype(jnp.bfloat16)
        w2 = jax.lax.dot_general(
            onehot, mw_ref[...], (((1,), (0,)), ((), ())),
            preferred_element_type=jnp.float32)
        mt = mt_ref[j]
        cs1 = jnp.sum(jnp.abs(mp_ref[j] - mt), axis=0, keepdims=True)
        cs2 = jnp.sum(jnp.abs(pmp_ref[j] - mt), axis=0, keepdims=True)
        acc1_ref[...] += cs1 * w2
        acc2_ref[...] += cs2 * w2

    @pl.when(i == B // BB - 1)
    def _final():
        n_mel = sacc_ref[6] * M
        mel_loss = jnp.sum(acc1_ref[...]) / n_mel
        postnet_loss = jnp.sum(acc2_ref[...]) / n_mel
        n_src = sacc_ref[3]
        pitch_loss = sacc_ref[0] / n_src
        energy_loss = sacc_ref[1] / n_src
        dur_loss = sacc_ref[2] / n_src
        pause_loss = (sacc_ref[4] / (B * S) + 100.0 * 0.5 * sacc_ref[5] / B) / S
        pause_w = pause_loss * 0.7
        out_ref[1] = mel_loss
        out_ref[2] = postnet_loss
        out_ref[3] = pitch_loss
        out_ref[4] = energy_loss
        out_ref[5] = dur_loss
        out_ref[6] = pause_w
        out_ref[0] = (mel_loss + postnet_loss + dur_loss + pitch_loss +
                      energy_loss + pause_w)


def kernel(mel_targets, pitch_targets, energy_targets, pause_targets,
           mel_predictions, postnet_mel_predictions, pitch_predictions,
           energy_predictions, log_duration_predictions, pause_predictions,
           duration_targets, src_masks, mel_masks):
    mt3 = jnp.transpose(mel_targets, (0, 2, 1))
    mp3 = jnp.transpose(mel_predictions, (0, 2, 1))
    pmp3 = jnp.transpose(postnet_mel_predictions, (0, 2, 1))
    mw = jnp.logical_not(mel_masks).astype(jnp.bfloat16)
    sw = jnp.logical_not(src_masks).astype(jnp.float32)

    mel_spec = pl.BlockSpec((BB, M, T), lambda i: (i, 0, 0))
    full2d = pl.BlockSpec((B, T), lambda i: (0, 0))
    small_spec = pl.BlockSpec((B, S), lambda i: (0, 0))

    out = pl.pallas_call(
        _body,
        grid=(B // BB,),
        in_specs=[mel_spec, mel_spec, mel_spec, full2d] + [small_spec] * 9,
        out_specs=pl.BlockSpec(memory_space=pltpu.SMEM),
        out_shape=jax.ShapeDtypeStruct((8,), jnp.float32),
        scratch_shapes=[pltpu.VMEM((1, T), jnp.float32),
                        pltpu.VMEM((1, T), jnp.float32),
                        pltpu.SMEM((8,), jnp.float32)],
    )(mt3, mp3, pmp3, mw,
      pitch_targets, pitch_predictions,
      energy_targets, energy_predictions,
      log_duration_predictions, duration_targets,
      pause_targets, pause_predictions, sw)

    return (out[0], out[1], out[2], out[3], out[4], out[5], out[6])
